# baseline (speedup 1.0000x reference)
import functools

import jax
import jax.numpy as jnp
from jax import lax
from jax.experimental import pallas as pl
from jax.experimental.pallas import tpu as pltpu
from jax.experimental.pallas import tpu_sc as plsc

N_NODES = 50000
HIDDEN = 512
N_GRAPHS = 256
LANES = 16
NC = 2
NS = 16
NW = NC * NS
GPW = N_GRAPHS // NW
RCH = 32
NBUF = 4
NV = HIDDEN // (2 * LANES)
NEG_BIG = -3.4028235e38


def _lower_bound(batch_v, target):
    def body(_, lohi):
        lo, hi = lohi
        notdone = lo < hi
        mid = jnp.minimum((lo + hi) // 2, N_NODES - 1)
        v = batch_v[pl.ds(mid, LANES)][0]
        less = jnp.logical_and(v < target, notdone)
        lo = jnp.where(less, mid + 1, lo)
        hi = jnp.where(jnp.logical_and(notdone, jnp.logical_not(less)), mid, hi)
        return lo, hi

    lo, _ = lax.fori_loop(0, 16, body, (jnp.int32(0), jnp.int32(N_NODES)))
    return lo


@functools.partial(
    pl.kernel,
    out_type=jax.ShapeDtypeStruct((N_GRAPHS, HIDDEN), jnp.float32),
    mesh=plsc.VectorSubcoreMesh(core_axis_name="c", subcore_axis_name="s"),
    scratch_types=[
        pltpu.VMEM((N_NODES + LANES,), jnp.int32),
        pltpu.VMEM((NBUF, RCH, HIDDEN), jnp.float32),
        pltpu.VMEM((GPW, 2 * HIDDEN), jnp.float32),
        pltpu.VMEM((3 * LANES,), jnp.float32),
        pltpu.SMEM((GPW + 1,), jnp.int32),
        pltpu.SemaphoreType.DMA((NBUF,)),
    ],
)
def _sc_readout(x_hbm, batch_hbm, wrep_hbm, out_hbm,
                batch_v, xbuf_v, acc_v, wbuf_v, sb_s, sems):
    cid = lax.axis_index("c")
    sid = lax.axis_index("s")
    wid = sid * NC + cid
    g0 = wid * GPW

    pltpu.sync_copy(batch_hbm, batch_v.at[pl.ds(0, N_NODES)])
    pltpu.sync_copy(wrep_hbm, wbuf_v)

    w0 = wbuf_v[pl.ds(0, LANES)]
    w1 = wbuf_v[pl.ds(LANES, LANES)]
    w2 = wbuf_v[pl.ds(2 * LANES, LANES)]

    for i in range(GPW + 1):
        sb_s[i] = _lower_bound(batch_v, g0 + i)

    zero = jnp.zeros((LANES,), jnp.float32)
    ninf = jnp.full((LANES,), NEG_BIG, jnp.float32)

    def init_body(gi, carry):
        for k in range(2 * NV):
            acc_v[gi, pl.ds(k * LANES, LANES)] = zero
            acc_v[gi, pl.ds(HIDDEN + k * LANES, LANES)] = ninf
        return carry

    lax.fori_loop(0, GPW, init_body, jnp.int32(0))

    r_lo = sb_s[0]
    r_hi = sb_s[GPW]
    a0 = (r_lo // 8) * 8
    nch = (r_hi - a0 + RCH - 1) // RCH

    def _issue(c):
        wstart = pl.multiple_of(jnp.minimum(a0 + c * RCH, N_NODES - RCH), 8)
        b = lax.rem(c, NBUF)
        pltpu.async_copy(x_hbm.at[pl.ds(wstart, RCH)], xbuf_v.at[b], sems.at[b])

    @pl.when(r_hi > r_lo)
    def _pipeline():
        for j in range(NBUF - 1):
            @pl.when(j < nch)
            def _(j=j):
                _issue(jnp.int32(j))

        def chunk_body(c, carry):
            b = lax.rem(c, NBUF)
            chunk_lo = a0 + c * RCH
            wstart = jnp.minimum(chunk_lo, N_NODES - RCH)
            pltpu.make_async_copy(
                x_hbm.at[pl.ds(0, RCH)], xbuf_v.at[b], sems.at[b]).wait()

            @pl.when(c + NBUF - 1 < nch)
            def _():
                _issue(c + NBUF - 1)

            def graph_body(gi, carry2):
                glo = sb_s[gi]
                ghi = sb_s[gi + 1]
                lo = jnp.maximum(glo, chunk_lo)
                hi = jnp.minimum(ghi, chunk_lo + RCH)

                @pl.when(hi > lo)
                def _():
                    off = lo - wstart
                    rows = hi - lo
                    for h in range(2):
                        base = h * NV * LANES
                        accs = tuple(
                            [acc_v[gi, pl.ds(base + k * LANES, LANES)]
                             for k in range(NV)]
                            + [acc_v[gi, pl.ds(HIDDEN + base + k * LANES, LANES)]
                               for k in range(NV)])

                        def row_body(r, a, off=off, base=base):
                            vs = [xbuf_v[b, off + r, pl.ds(base + k * LANES, LANES)]
                                  for k in range(NV)]
                            return tuple(
                                [a[k] + vs[k] for k in range(NV)]
                                + [jnp.maximum(a[NV + k], vs[k])
                                   for k in range(NV)])

                        accs = lax.fori_loop(0, rows, row_body, accs)
                        for k in range(NV):
                            acc_v[gi, pl.ds(base + k * LANES, LANES)] = accs[k]
                            acc_v[gi, pl.ds(HIDDEN + base + k * LANES, LANES)] = (
                                accs[NV + k])

                return carry2

            lax.fori_loop(0, GPW, graph_body, jnp.int32(0))
            return carry

        lax.fori_loop(0, nch, chunk_body, jnp.int32(0))

    for gi in range(GPW):
        nrows = sb_s[gi + 1] - sb_s[gi]
        cnt_v = jnp.full((LANES,), nrows.astype(jnp.float32))
        denom = jnp.maximum(cnt_v, jnp.ones((LANES,), jnp.float32))
        scale_a = w0 / denom + w2
        ind = jnp.full((LANES,), jnp.minimum(nrows, 1).astype(jnp.float32))
        w1_eff = w1 * ind
        for k in range(2 * NV):
            s = acc_v[gi, pl.ds(k * LANES, LANES)]
            m = acc_v[gi, pl.ds(HIDDEN + k * LANES, LANES)]
            acc_v[gi, pl.ds(k * LANES, LANES)] = scale_a * s + w1_eff * m

    pltpu.sync_copy(acc_v.at[pl.ds(0, GPW), pl.ds(0, HIDDEN)],
                    out_hbm.at[pl.ds(pl.multiple_of(g0, 8), GPW)])


def kernel(x, batch, weights):
    wrep = jnp.repeat(weights, LANES)
    return _sc_readout(x, batch, wrep)


# speedup vs baseline: 1.0261x; 1.0261x over previous
import functools

import jax
import jax.numpy as jnp
from jax import lax
from jax.experimental import pallas as pl
from jax.experimental.pallas import tpu as pltpu
from jax.experimental.pallas import tpu_sc as plsc

N_NODES = 50000
HIDDEN = 512
N_GRAPHS = 256
LANES = 16
NC = 2
NS = 16
NW = NC * NS
GPW = N_GRAPHS // NW
RCH = 40
NBUF = 3
NV = HIDDEN // (2 * LANES)
NEG_BIG = -3.4028235e38


def _lower_bound(batch_v, target):
    def body(_, lohi):
        lo, hi = lohi
        notdone = lo < hi
        mid = jnp.minimum((lo + hi) // 2, N_NODES - 1)
        v = batch_v[pl.ds(mid, LANES)][0]
        less = jnp.logical_and(v < target, notdone)
        lo = jnp.where(less, mid + 1, lo)
        hi = jnp.where(jnp.logical_and(notdone, jnp.logical_not(less)), mid, hi)
        return lo, hi

    lo, _ = lax.fori_loop(0, 16, body, (jnp.int32(0), jnp.int32(N_NODES)))
    return lo


@functools.partial(
    pl.kernel,
    out_type=jax.ShapeDtypeStruct((N_GRAPHS, HIDDEN), jnp.float32),
    mesh=plsc.VectorSubcoreMesh(core_axis_name="c", subcore_axis_name="s"),
    scratch_types=[
        pltpu.VMEM((N_NODES + LANES,), jnp.int32),
        pltpu.VMEM((NBUF, RCH, HIDDEN), jnp.float32),
        pltpu.VMEM((GPW, 2 * HIDDEN), jnp.float32),
        pltpu.VMEM((3 * LANES,), jnp.float32),
        pltpu.SMEM((GPW + 1,), jnp.int32),
        pltpu.SemaphoreType.DMA((NBUF,)),
    ],
)
def _sc_readout(x_hbm, batch_hbm, wrep_hbm, out_hbm,
                batch_v, xbuf_v, acc_v, wbuf_v, sb_s, sems):
    cid = lax.axis_index("c")
    sid = lax.axis_index("s")
    wid = sid * NC + cid
    g0 = wid * GPW

    pltpu.sync_copy(batch_hbm, batch_v.at[pl.ds(0, N_NODES)])
    pltpu.sync_copy(wrep_hbm, wbuf_v)

    w0 = wbuf_v[pl.ds(0, LANES)]
    w1 = wbuf_v[pl.ds(LANES, LANES)]
    w2 = wbuf_v[pl.ds(2 * LANES, LANES)]

    for i in range(GPW + 1):
        sb_s[i] = _lower_bound(batch_v, g0 + i)

    zero = jnp.zeros((LANES,), jnp.float32)
    ninf = jnp.full((LANES,), NEG_BIG, jnp.float32)

    def init_body(gi, carry):
        for k in range(2 * NV):
            acc_v[gi, pl.ds(k * LANES, LANES)] = zero
            acc_v[gi, pl.ds(HIDDEN + k * LANES, LANES)] = ninf
        return carry

    lax.fori_loop(0, GPW, init_body, jnp.int32(0))

    r_lo = sb_s[0]
    r_hi = sb_s[GPW]
    a0 = (r_lo // 8) * 8
    nch = (r_hi - a0 + RCH - 1) // RCH

    def _issue(c):
        wstart = pl.multiple_of(jnp.minimum(a0 + c * RCH, N_NODES - RCH), 8)
        b = lax.rem(c, NBUF)
        pltpu.async_copy(x_hbm.at[pl.ds(wstart, RCH)], xbuf_v.at[b], sems.at[b])

    @pl.when(r_hi > r_lo)
    def _pipeline():
        for j in range(NBUF - 1):
            @pl.when(j < nch)
            def _(j=j):
                _issue(jnp.int32(j))

        def chunk_body(c, carry):
            b = lax.rem(c, NBUF)
            chunk_lo = a0 + c * RCH
            wstart = jnp.minimum(chunk_lo, N_NODES - RCH)
            pltpu.make_async_copy(
                x_hbm.at[pl.ds(0, RCH)], xbuf_v.at[b], sems.at[b]).wait()

            @pl.when(c + NBUF - 1 < nch)
            def _():
                _issue(c + NBUF - 1)

            def graph_body(gi, carry2):
                glo = sb_s[gi]
                ghi = sb_s[gi + 1]
                lo = jnp.maximum(glo, chunk_lo)
                hi = jnp.minimum(ghi, chunk_lo + RCH)

                @pl.when(hi > lo)
                def _():
                    off = lo - wstart
                    rows = hi - lo
                    for h in range(2):
                        base = h * NV * LANES
                        accs = tuple(
                            [acc_v[gi, pl.ds(base + k * LANES, LANES)]
                             for k in range(NV)]
                            + [acc_v[gi, pl.ds(HIDDEN + base + k * LANES, LANES)]
                               for k in range(NV)])

                        def row_body(r, a, off=off, base=base):
                            vs = [xbuf_v[b, off + r, pl.ds(base + k * LANES, LANES)]
                                  for k in range(NV)]
                            return tuple(
                                [a[k] + vs[k] for k in range(NV)]
                                + [jnp.maximum(a[NV + k], vs[k])
                                   for k in range(NV)])

                        accs = lax.fori_loop(0, rows, row_body, accs)
                        for k in range(NV):
                            acc_v[gi, pl.ds(base + k * LANES, LANES)] = accs[k]
                            acc_v[gi, pl.ds(HIDDEN + base + k * LANES, LANES)] = (
                                accs[NV + k])

                return carry2

            lax.fori_loop(0, GPW, graph_body, jnp.int32(0))
            return carry

        lax.fori_loop(0, nch, chunk_body, jnp.int32(0))

    for gi in range(GPW):
        nrows = sb_s[gi + 1] - sb_s[gi]
        cnt_v = jnp.full((LANES,), nrows.astype(jnp.float32))
        denom = jnp.maximum(cnt_v, jnp.ones((LANES,), jnp.float32))
        scale_a = w0 / denom + w2
        ind = jnp.full((LANES,), jnp.minimum(nrows, 1).astype(jnp.float32))
        w1_eff = w1 * ind
        for k in range(2 * NV):
            s = acc_v[gi, pl.ds(k * LANES, LANES)]
            m = acc_v[gi, pl.ds(HIDDEN + k * LANES, LANES)]
            acc_v[gi, pl.ds(k * LANES, LANES)] = scale_a * s + w1_eff * m

    pltpu.sync_copy(acc_v.at[pl.ds(0, GPW), pl.ds(0, HIDDEN)],
                    out_hbm.at[pl.ds(pl.multiple_of(g0, 8), GPW)])


def kernel(x, batch, weights):
    wrep = jnp.repeat(weights, LANES)
    return _sc_readout(x, batch, wrep)
